# Initial kernel scaffold; baseline (speedup 1.0000x reference)
import jax
import jax.numpy as jnp
from jax.experimental import pallas as pl


def kernel(indices, species_table):
    raise NotImplementedError("write your pallas kernel here")


# speedup vs baseline: 2.7539x; 2.7539x over previous
import functools

import jax
import jax.numpy as jnp
from jax import lax
from jax.experimental import pallas as pl
from jax.experimental.pallas import tpu as pltpu
from jax.experimental.pallas import tpu_sc as plsc

EMBED_DIM = 128
NC = 2
NS = 16
NW = NC * NS
CHUNK = 128


@functools.lru_cache(maxsize=None)
def _make_kernel(B: int):
    assert B % (NW * CHUNK) == 0
    n_chunks = B // (NW * CHUNK)
    mesh = plsc.VectorSubcoreMesh(core_axis_name="c", subcore_axis_name="s")

    @functools.partial(
        pl.kernel,
        mesh=mesh,
        out_type=jax.ShapeDtypeStruct((B, EMBED_DIM), jnp.float32),
        scratch_types=[
            pltpu.VMEM((n_chunks, CHUNK), jnp.int32),
            pltpu.VMEM((CHUNK, EMBED_DIM), jnp.float32),
            pltpu.SemaphoreType.DMA,
        ],
    )
    def k(idx_hbm, table_hbm, out_hbm, idx_v, rows_v, sem):
        wid = lax.axis_index("s") * NC + lax.axis_index("c")
        pltpu.sync_copy(idx_hbm.at[wid], idx_v)

        def body(j, carry):
            pltpu.async_copy(table_hbm.at[idx_v.at[j]], rows_v, sem).wait()
            base = (wid * n_chunks + j) * CHUNK
            pltpu.sync_copy(rows_v, out_hbm.at[pl.ds(base, CHUNK)])
            return carry

        lax.fori_loop(0, n_chunks, body, 0)

    return k


@jax.jit
def kernel(indices, species_table):
    B, H = indices.shape
    n = B * H
    idx3d = indices.reshape(NW, n // (NW * CHUNK), CHUNK).astype(jnp.int32)
    out = _make_kernel(n)(idx3d, species_table)
    return out.reshape(B, H, EMBED_DIM)


# speedup vs baseline: 2.8997x; 1.0530x over previous
import functools

import jax
import jax.numpy as jnp
from jax import lax
from jax.experimental import pallas as pl
from jax.experimental.pallas import tpu as pltpu
from jax.experimental.pallas import tpu_sc as plsc

EMBED_DIM = 128
NC = 2
NS = 16
NW = NC * NS
CHUNK = 128
NBUF = 6


@functools.lru_cache(maxsize=None)
def _make_kernel(B: int):
    assert B % (NW * CHUNK) == 0
    n_chunks = B // (NW * CHUNK)
    assert n_chunks > NBUF
    mesh = plsc.VectorSubcoreMesh(core_axis_name="c", subcore_axis_name="s")

    @functools.partial(
        pl.kernel,
        mesh=mesh,
        out_type=jax.ShapeDtypeStruct((B, EMBED_DIM), jnp.float32),
        scratch_types=[
            pltpu.VMEM((n_chunks, CHUNK), jnp.int32),
            pltpu.VMEM((NBUF, CHUNK, EMBED_DIM), jnp.float32),
            pltpu.SemaphoreType.DMA((NBUF,)),
            pltpu.SemaphoreType.DMA((NBUF,)),
        ],
    )
    def k(idx_hbm, table_hbm, out_hbm, idx_v, rows_v, gsem, ssem):
        wid = lax.axis_index("s") * NC + lax.axis_index("c")
        pltpu.sync_copy(idx_hbm.at[wid], idx_v)
        base = wid * n_chunks

        def fire_gather(j):
            b = j % NBUF
            pltpu.async_copy(table_hbm.at[idx_v.at[j]], rows_v.at[b], gsem.at[b])

        def drain_gather(j):
            b = j % NBUF
            pltpu.make_async_copy(
                table_hbm.at[idx_v.at[j]], rows_v.at[b], gsem.at[b]
            ).wait()

        def fire_scatter(j):
            b = j % NBUF
            pltpu.async_copy(
                rows_v.at[b], out_hbm.at[pl.ds((base + j) * CHUNK, CHUNK)], ssem.at[b]
            )

        def drain_scatter(j):
            b = j % NBUF
            pltpu.make_async_copy(
                rows_v.at[b], out_hbm.at[pl.ds((base + j) * CHUNK, CHUNK)], ssem.at[b]
            ).wait()

        fire_gather(0)
        fire_gather(1)

        def body(j, carry):
            drain_gather(j)
            fire_scatter(j)
            @pl.when(j >= NBUF - 2)
            def _():
                drain_scatter(j - (NBUF - 2))

            @pl.when(j + 2 < n_chunks)
            def _():
                fire_gather(j + 2)

            return carry

        lax.fori_loop(0, n_chunks, body, 0)
        for t in range(NBUF - 2):
            drain_scatter(n_chunks - (NBUF - 2) + t)

    return k


@jax.jit
def kernel(indices, species_table):
    B, H = indices.shape
    n = B * H
    idx3d = indices.reshape(NW, n // (NW * CHUNK), CHUNK).astype(jnp.int32)
    out = _make_kernel(n)(idx3d, species_table)
    return out.reshape(B, H, EMBED_DIM)


# speedup vs baseline: 4.7631x; 1.6426x over previous
import functools

import jax
import jax.numpy as jnp
from jax import lax
from jax.experimental import pallas as pl
from jax.experimental.pallas import tpu as pltpu
from jax.experimental.pallas import tpu_sc as plsc

EMBED_DIM = 128
NC = 2
NS = 16
NW = NC * NS
NB_ROWS = 4
NBUF = 4


@functools.lru_cache(maxsize=None)
def _make_kernel(B: int, H: int):
    assert B % (NW * NB_ROWS) == 0 and H <= 128
    rows_per_worker = B // NW
    n_slots = rows_per_worker // NB_ROWS
    assert n_slots > NBUF
    mesh = plsc.VectorSubcoreMesh(core_axis_name="c", subcore_axis_name="s")

    @functools.partial(
        pl.kernel,
        mesh=mesh,
        out_type=jax.ShapeDtypeStruct((B, H, EMBED_DIM), jnp.float32),
        scratch_types=[
            pltpu.VMEM((rows_per_worker, H), jnp.int32),
            pltpu.VMEM((NBUF, NB_ROWS, H, EMBED_DIM), jnp.float32),
            pltpu.SemaphoreType.DMA((NBUF,)),
            pltpu.SemaphoreType.DMA((NBUF,)),
        ],
    )
    def k(idx_hbm, table_hbm, out_hbm, idx_v, rows_v, gsem, ssem):
        wid = lax.axis_index("s") * NC + lax.axis_index("c")
        pltpu.sync_copy(idx_hbm.at[wid], idx_v)
        row0 = wid * rows_per_worker

        def fire_gathers(c):
            b = c % NBUF
            for s in range(NB_ROWS):
                pltpu.async_copy(
                    table_hbm.at[idx_v.at[c * NB_ROWS + s]],
                    rows_v.at[b, s],
                    gsem.at[b],
                )

        def drain_gathers(c):
            b = c % NBUF
            for s in range(NB_ROWS):
                pltpu.make_async_copy(
                    table_hbm.at[idx_v.at[c * NB_ROWS + s]],
                    rows_v.at[b, s],
                    gsem.at[b],
                ).wait()

        def fire_scatter(c):
            b = c % NBUF
            pltpu.async_copy(
                rows_v.at[b],
                out_hbm.at[pl.ds(row0 + c * NB_ROWS, NB_ROWS)],
                ssem.at[b],
            )

        def drain_scatter(c):
            b = c % NBUF
            pltpu.make_async_copy(
                rows_v.at[b],
                out_hbm.at[pl.ds(row0 + c * NB_ROWS, NB_ROWS)],
                ssem.at[b],
            ).wait()

        fire_gathers(0)
        fire_gathers(1)

        def body(c, carry):
            drain_gathers(c)
            fire_scatter(c)
            @pl.when(c >= NBUF - 2)
            def _():
                drain_scatter(c - (NBUF - 2))

            @pl.when(c + 2 < n_slots)
            def _():
                fire_gathers(c + 2)

            return carry

        lax.fori_loop(0, n_slots, body, 0)
        for t in range(NBUF - 2):
            drain_scatter(n_slots - (NBUF - 2) + t)

    return k


@jax.jit
def kernel(indices, species_table):
    B, H = indices.shape
    idx3d = indices.reshape(NW, B // NW, H).astype(jnp.int32)
    return _make_kernel(B, H)(idx3d, species_table)
